# Initial kernel scaffold; baseline (speedup 1.0000x reference)
import jax
import jax.numpy as jnp
from jax.experimental import pallas as pl


def kernel(sequence, segment_label, token_table, segment_table, pe, a_2, b_2):
    raise NotImplementedError("write your pallas kernel here")


# speedup vs baseline: 1.0324x; 1.0324x over previous
import jax
import jax.numpy as jnp
from jax import lax
from jax.experimental import pallas as pl
from jax.experimental.pallas import tpu as pltpu
from jax.experimental.pallas import tpu_sc as plsc

E = 64
EC = E // 16
NPOS = 200
NC = 2
NS = 16
NW = NC * NS
CHUNK = 640
SUB = 128
NSUB = CHUNK // SUB
GROUP = 16


def _perm(v, idx):
    return v.at[idx].get(mode="promise_in_bounds")


def _sc_embed_ln(seq1d, lbl, token_table, seg_table, pe2d, a2, b2, n_rows):
    rows_per_w = n_rows // NW
    n_chunks = rows_per_w // CHUNK
    mesh = plsc.VectorSubcoreMesh(core_axis_name="c", subcore_axis_name="s")

    def body(seq_hbm, lbl_hbm, tab_hbm, seg_hbm, pe_hbm, a2_hbm, b2_hbm,
             out_hbm, idx_s, lblv, xbuf, pebuf, segbuf, a2v_m, b2v_m, sem):
        wid = lax.axis_index("s") * NC + lax.axis_index("c")
        pltpu.sync_copy(pe_hbm, pebuf)
        pltpu.sync_copy(seg_hbm, segbuf)
        pltpu.sync_copy(a2_hbm, a2v_m)
        pltpu.sync_copy(b2_hbm, b2v_m)
        a2v = [a2v_m[pl.ds(c * 16, 16)] for c in range(EC)]
        b2v = [b2v_m[pl.ds(c * 16, 16)] for c in range(EC)]
        seg0 = [segbuf[0, pl.ds(c * 16, 16)] for c in range(EC)]
        dseg = [segbuf[1, pl.ds(c * 16, 16)] - seg0[c] for c in range(EC)]
        iota = jnp.arange(16, dtype=jnp.int32)
        bfly = [iota ^ m for m in (8, 4, 2, 1)]
        lanes = [jnp.full((16,), u, dtype=jnp.int32) for u in range(GROUP)]

        def hsum(v):
            for bidx in bfly:
                v = v + _perm(v, bidx)
            return v

        def process_row(r, svf, base):
            l = lax.rem(base + r, NPOS)
            xp = []
            for c in range(EC):
                x = xbuf[r, pl.ds(c * 16, 16)]
                p = pebuf[l, pl.ds(c * 16, 16)]
                sg = svf * dseg[c] + seg0[c]
                xp.append((x + p) + sg)
            s1v = (xp[0] + xp[1]) + (xp[2] + xp[3])
            s2v = (xp[0] * xp[0] + xp[1] * xp[1]) + (xp[2] * xp[2] + xp[3] * xp[3])
            st = hsum(s1v)
            sst = hsum(s2v)
            mean = st * (1.0 / E)
            var = (sst - st * mean) * (1.0 / (E - 1))
            v = jnp.maximum(var, 1e-20)
            i = lax.bitcast_convert_type(v, jnp.int32)
            i = jnp.int32(0x5F3759DF) - (i >> 1)
            rsq = lax.bitcast_convert_type(i, jnp.float32)
            rsq = rsq * (1.5 - 0.5 * v * rsq * rsq)
            rsq = rsq * (1.5 - 0.5 * v * rsq * rsq)
            d = v * rsq + 1e-6
            inv = rsq * (2.0 - d * rsq)
            for c in range(EC):
                y = (xp[c] - mean) * inv * a2v[c] + b2v[c]
                xbuf[r, pl.ds(c * 16, 16)] = y

        def chunk_body(ch, carry):
            base = wid * rows_per_w + ch * CHUNK
            pltpu.sync_copy(seq_hbm.at[pl.ds(base, CHUNK)], idx_s)
            pltpu.sync_copy(lbl_hbm.at[pl.ds(base, CHUNK)], lblv)
            copies = [
                pltpu.async_copy(tab_hbm.at[idx_s.at[pl.ds(j * SUB, SUB)]],
                                 xbuf.at[pl.ds(j * SUB, SUB)], sem)
                for j in range(NSUB)
            ]
            for cp in copies:
                cp.wait()

            def row_body(g, carry2):
                lblf = lblv[pl.ds(g * GROUP, GROUP)].astype(jnp.float32)
                for u in range(GROUP):
                    process_row(g * GROUP + u, _perm(lblf, lanes[u]), base)
                return carry2

            lax.fori_loop(0, CHUNK // GROUP, row_body, 0)
            pltpu.sync_copy(xbuf, out_hbm.at[pl.ds(base, CHUNK)])
            return carry

        lax.fori_loop(0, n_chunks, chunk_body, 0)

    call = pl.kernel(
        body,
        out_type=jax.ShapeDtypeStruct((n_rows, E), jnp.float32),
        mesh=mesh,
        scratch_types=[
            pltpu.VMEM((CHUNK,), jnp.int32),
            pltpu.VMEM((CHUNK,), jnp.int32),
            pltpu.VMEM((CHUNK, E), jnp.float32),
            pltpu.VMEM((NPOS, E), jnp.float32),
            pltpu.VMEM((2, E), jnp.float32),
            pltpu.VMEM((E,), jnp.float32),
            pltpu.VMEM((E,), jnp.float32),
            pltpu.SemaphoreType.DMA,
        ],
        compiler_params=pltpu.CompilerParams(use_tc_tiling_on_sc=False),
    )
    return call(seq1d, lbl, token_table, seg_table, pe2d, a2, b2)


def kernel(sequence, segment_label, token_table, segment_table, pe, a_2, b_2):
    b, l = sequence.shape
    n_rows = b * l
    seq1d = sequence.reshape(n_rows)
    lbl = segment_label.reshape(n_rows)
    pe2d = pe[0, :l, :]
    out = _sc_embed_ln(seq1d, lbl, token_table, segment_table, pe2d,
                       a_2, b_2, n_rows)
    return out.reshape(b, l, E)


# speedup vs baseline: 1.0327x; 1.0002x over previous
import jax
import jax.numpy as jnp
from jax import lax
from jax.experimental import pallas as pl
from jax.experimental.pallas import tpu as pltpu
from jax.experimental.pallas import tpu_sc as plsc

E = 64
EC = E // 16
NPOS = 200
NC = 2
NS = 16
NW = NC * NS
CHUNK = 640
SUB = 128
NSUB = CHUNK // SUB
GROUP = 16


def _perm(v, idx):
    return v.at[idx].get(mode="promise_in_bounds")


def _sc_embed_ln(seq1d, lbl, token_table, seg_table, pe2d, a2, b2, n_rows):
    rows_per_w = n_rows // NW
    n_chunks = rows_per_w // CHUNK
    mesh = plsc.VectorSubcoreMesh(core_axis_name="c", subcore_axis_name="s")

    def body(seq_hbm, lbl_hbm, tab_hbm, seg_hbm, pe_hbm, a2_hbm, b2_hbm,
             out_hbm, idx_s, lblv, xbuf, ybuf, pebuf, segbuf, a2v_m, b2v_m, sem):
        wid = lax.axis_index("s") * NC + lax.axis_index("c")
        pltpu.sync_copy(pe_hbm, pebuf)
        pltpu.sync_copy(seg_hbm, segbuf)
        pltpu.sync_copy(a2_hbm, a2v_m)
        pltpu.sync_copy(b2_hbm, b2v_m)
        a2v = [a2v_m[pl.ds(c * 16, 16)] for c in range(EC)]
        b2v = [b2v_m[pl.ds(c * 16, 16)] for c in range(EC)]
        seg0 = [segbuf[0, pl.ds(c * 16, 16)] for c in range(EC)]
        dseg = [segbuf[1, pl.ds(c * 16, 16)] - seg0[c] for c in range(EC)]
        iota = jnp.arange(16, dtype=jnp.int32)
        bfly = [iota ^ m for m in (8, 4, 2, 1)]
        lanes = [jnp.full((16,), u, dtype=jnp.int32) for u in range(GROUP)]

        def hsum(v):
            for bidx in bfly:
                v = v + _perm(v, bidx)
            return v

        def process_row(r, svf, base):
            l = lax.rem(base + r, NPOS)
            xp = []
            for c in range(EC):
                x = xbuf[r, pl.ds(c * 16, 16)]
                p = pebuf[l, pl.ds(c * 16, 16)]
                sg = svf * dseg[c] + seg0[c]
                xp.append((x + p) + sg)
            s1v = (xp[0] + xp[1]) + (xp[2] + xp[3])
            s2v = (xp[0] * xp[0] + xp[1] * xp[1]) + (xp[2] * xp[2] + xp[3] * xp[3])
            st = hsum(s1v)
            sst = hsum(s2v)
            mean = st * (1.0 / E)
            var = (sst - st * mean) * (1.0 / (E - 1))
            v = jnp.maximum(var, 1e-20)
            i = lax.bitcast_convert_type(v, jnp.int32)
            i = jnp.int32(0x5F3759DF) - (i >> 1)
            rsq = lax.bitcast_convert_type(i, jnp.float32)
            rsq = rsq * (1.5 - 0.5 * v * rsq * rsq)
            rsq = rsq * (1.5 - 0.5 * v * rsq * rsq)
            d = v * rsq + 1e-6
            inv = rsq * (2.0 - d * rsq)
            for c in range(EC):
                y = (xp[c] - mean) * inv * a2v[c] + b2v[c]
                ybuf[r, pl.ds(c * 16, 16)] = y

        def chunk_body(ch, carry):
            base = wid * rows_per_w + ch * CHUNK
            pltpu.sync_copy(seq_hbm.at[pl.ds(base, CHUNK)], idx_s)
            pltpu.sync_copy(lbl_hbm.at[pl.ds(base, CHUNK)], lblv)
            copies = [
                pltpu.async_copy(tab_hbm.at[idx_s.at[pl.ds(j * SUB, SUB)]],
                                 xbuf.at[pl.ds(j * SUB, SUB)], sem)
                for j in range(NSUB)
            ]
            for cp in copies:
                cp.wait()

            def row_body(g, carry2):
                lblf = lblv[pl.ds(g * GROUP, GROUP)].astype(jnp.float32)
                for u in range(GROUP):
                    process_row(g * GROUP + u, _perm(lblf, lanes[u]), base)
                return carry2

            lax.fori_loop(0, CHUNK // GROUP, row_body, 0)
            pltpu.sync_copy(ybuf, out_hbm.at[pl.ds(base, CHUNK)])
            return carry

        lax.fori_loop(0, n_chunks, chunk_body, 0)

    call = pl.kernel(
        body,
        out_type=jax.ShapeDtypeStruct((n_rows, E), jnp.float32),
        mesh=mesh,
        scratch_types=[
            pltpu.VMEM((CHUNK,), jnp.int32),
            pltpu.VMEM((CHUNK,), jnp.int32),
            pltpu.VMEM((CHUNK, E), jnp.float32),
            pltpu.VMEM((CHUNK, E), jnp.float32),
            pltpu.VMEM((NPOS, E), jnp.float32),
            pltpu.VMEM((2, E), jnp.float32),
            pltpu.VMEM((E,), jnp.float32),
            pltpu.VMEM((E,), jnp.float32),
            pltpu.SemaphoreType.DMA,
        ],
        compiler_params=pltpu.CompilerParams(use_tc_tiling_on_sc=False),
    )
    return call(seq1d, lbl, token_table, seg_table, pe2d, a2, b2)


def kernel(sequence, segment_label, token_table, segment_table, pe, a_2, b_2):
    b, l = sequence.shape
    n_rows = b * l
    seq1d = sequence.reshape(n_rows)
    lbl = segment_label.reshape(n_rows)
    pe2d = pe[0, :l, :]
    out = _sc_embed_ln(seq1d, lbl, token_table, segment_table, pe2d,
                       a_2, b_2, n_rows)
    return out.reshape(b, l, E)
